# baseline (speedup 1.0000x reference)
import functools

import jax
import jax.numpy as jnp
from jax import lax
from jax.experimental import pallas as pl
from jax.experimental.pallas import tpu as pltpu
from jax.experimental.pallas import tpu_sc as plsc

_BATCH = 16384
_DIM = 32
_DEPTH = 8


def _make_gather(batch, dim):
    info = plsc.get_sparse_core_info()
    nc, ns = info.num_cores, info.num_subcores
    nw = nc * ns
    b_per_w = batch // nw
    n_groups = b_per_w // _DEPTH - 1
    mesh = plsc.VectorSubcoreMesh(core_axis_name="c", subcore_axis_name="s")

    scratch = [pltpu.VMEM((b_per_w,), jnp.int32),
               pltpu.VMEM((dim, b_per_w), jnp.float32)]
    scratch += [pltpu.VMEM((dim, 128), jnp.float32) for _ in range(_DEPTH)]
    scratch += [pltpu.SemaphoreType.DMA for _ in range(_DEPTH)]

    @functools.partial(
        pl.kernel,
        mesh=mesh,
        out_type=jax.ShapeDtypeStruct((dim, batch), jnp.float32),
        scratch_types=scratch,
        compiler_params=pltpu.CompilerParams(needs_layout_passes=False),
    )
    def gather_kernel(idx_hbm, table_hbm, out_hbm, idx_v, slab_v, *ring):
        bufs = ring[:_DEPTH]
        sems = ring[_DEPTH:]
        wid = lax.axis_index("s") * nc + lax.axis_index("c")
        base = wid * b_per_w
        pltpu.sync_copy(idx_hbm.at[pl.ds(base, b_per_w)], idx_v)

        rows0 = lax.iota(jnp.int32, 16)
        rows1 = rows0 + 16

        def read_id(j):
            chunk = idx_v[pl.ds(lax.shift_right_logical(j, 4) * 16, 16)]
            sel = jnp.where(rows0 == (j & 15), chunk, 0)
            return jnp.max(sel)

        def fire(r, j):
            tc = lax.shift_right_logical(read_id(j), 7)
            pltpu.async_copy(
                table_hbm.at[:, pl.ds(tc * 128, 128)], bufs[r], sems[r]
            )

        def wait(r):
            pltpu.make_async_copy(
                table_hbm.at[:, pl.ds(0, 128)], bufs[r], sems[r]
            ).wait()

        def extract(r, j):
            lane = read_id(j) & 127
            lane_v = jnp.zeros((16,), jnp.int32) + lane
            j_v = jnp.zeros((16,), jnp.int32) + j
            x0 = plsc.load_gather(bufs[r], [rows0, lane_v])
            x1 = plsc.load_gather(bufs[r], [rows1, lane_v])
            plsc.store_scatter(slab_v, [rows0, j_v], x0)
            plsc.store_scatter(slab_v, [rows1, j_v], x1)

        for r in range(_DEPTH):
            fire(r, r)

        def outer(g, carry):
            for r in range(_DEPTH):
                j = g * _DEPTH + r
                wait(r)
                extract(r, j)
                fire(r, j + _DEPTH)
            return carry

        lax.fori_loop(0, n_groups, outer, 0)

        for r in range(_DEPTH):
            j = n_groups * _DEPTH + r
            wait(r)
            extract(r, j)

        pltpu.sync_copy(slab_v, out_hbm.at[:, pl.ds(base, b_per_w)])

    return gather_kernel


_gather = _make_gather(_BATCH, _DIM)


def kernel(hero_ids, table):
    out_t = _gather(hero_ids.astype(jnp.int32), table.T)
    return out_t.T
